# Initial kernel scaffold; baseline (speedup 1.0000x reference)
import jax
import jax.numpy as jnp
from jax.experimental import pallas as pl


def kernel(coords, pairs, box, charges, prefac, cutoff):
    raise NotImplementedError("write your pallas kernel here")


# speedup vs baseline: 12.0479x; 12.0479x over previous
import functools

import jax
import jax.numpy as jnp
from jax import lax
from jax.experimental import pallas as pl
from jax.experimental.pallas import tpu as pltpu
from jax.experimental.pallas import tpu_sc as plsc

NC = 2
NS = 16
NW = NC * NS
L = 16

IDX_PER_CHUNK = 128
PAIRS_PER_CHUNK = IDX_PER_CHUNK // 2
CHUNKS_PER_BLK = 25
SEG = PAIRS_PER_CHUNK
BLK_IDX = CHUNKS_PER_BLK * IDX_PER_CHUNK
BLK_EXP = 4 * BLK_IDX


def _bf16r(x):
    b = plsc.bitcast(x, jnp.int32)
    lsb = lax.bitwise_and(lax.shift_right_logical(b, 16), 1)
    r = b + (lsb + 32767)
    r = lax.bitwise_and(r, jnp.int32(-65536))
    return plsc.bitcast(r, jnp.float32)


def _body(chunks_per_tile, n4, tab, idx, par, out,
          tab_sh, par_v, idx_v, exp_v, rows_v, acc_v, sem):
    cid = lax.axis_index("c")
    sid = lax.axis_index("s")
    wid = sid * NC + cid

    @pl.when(sid == 0)
    def _():
        pltpu.sync_copy(tab, tab_sh)

    pltpu.sync_copy(par, par_v)
    plsc.subcore_barrier()

    plo = par_v[pl.ds(0, L)]
    phi = par_v[pl.ds(8, L)]
    pe = lambda i: plo[i] if i < L else phi[i - 8]
    bi = [pe(i) for i in range(9)]
    bx = [pe(9 + i) for i in range(9)]
    cut2 = pe(18)
    icut = pe(19)

    iota = lax.iota(jnp.int32, L)
    iota2 = iota * 2

    blocks = chunks_per_tile // CHUNKS_PER_BLK
    tile_idx0 = wid * chunks_per_tile * IDX_PER_CHUNK

    def block(b, acc):
        pltpu.sync_copy(idx.at[pl.ds(tile_idx0 + b * BLK_IDX, BLK_IDX)], idx_v)

        def build(k, carry):
            for h in range(PAIRS_PER_CHUNK // L):
                off = k * IDX_PER_CHUNK + 2 * L * h
                asrc = plsc.load_gather(idx_v, [iota2 + off])
                adst = plsc.load_gather(idx_v, [iota2 + (off + 1)])
                s4 = asrc * 4
                d4 = adst * 4
                eb = k * (8 * SEG) + L * h
                exp_v[pl.ds(eb + 0 * SEG, L)] = s4
                exp_v[pl.ds(eb + 1 * SEG, L)] = d4
                exp_v[pl.ds(eb + 2 * SEG, L)] = s4 + 1
                exp_v[pl.ds(eb + 3 * SEG, L)] = d4 + 1
                exp_v[pl.ds(eb + 4 * SEG, L)] = s4 + 2
                exp_v[pl.ds(eb + 5 * SEG, L)] = d4 + 2
                exp_v[pl.ds(eb + 6 * SEG, L)] = s4 + 3
                exp_v[pl.ds(eb + 7 * SEG, L)] = d4 + 3
            return carry

        lax.fori_loop(0, CHUNKS_PER_BLK, build, 0)
        pltpu.async_copy(tab_sh.at[exp_v], rows_v, sem).wait()

        def chunk(k, a):
            cb = k * (8 * SEG)
            for g in range(PAIRS_PER_CHUNK // L):
                gb = L * g
                sx = rows_v[pl.ds(cb + 0 * SEG + gb, L)]
                dx = rows_v[pl.ds(cb + 1 * SEG + gb, L)]
                sy = rows_v[pl.ds(cb + 2 * SEG + gb, L)]
                dy = rows_v[pl.ds(cb + 3 * SEG + gb, L)]
                sz = rows_v[pl.ds(cb + 4 * SEG + gb, L)]
                dz = rows_v[pl.ds(cb + 5 * SEG + gb, L)]
                sq = rows_v[pl.ds(cb + 6 * SEG + gb, L)]
                dq = rows_v[pl.ds(cb + 7 * SEG + gb, L)]

                qx = _bf16r(sx - dx)
                qy = _bf16r(sy - dy)
                qz = _bf16r(sz - dz)
                dsx = qx * bi[0] + qy * bi[3] + qz * bi[6]
                dsy = qx * bi[1] + qy * bi[4] + qz * bi[7]
                dsz = qx * bi[2] + qy * bi[5] + qz * bi[8]
                tx = dsx + 0.5
                ty = dsy + 0.5
                tz = dsz + 0.5
                fx = tx.astype(jnp.int32).astype(jnp.float32)
                fy = ty.astype(jnp.int32).astype(jnp.float32)
                fz = tz.astype(jnp.int32).astype(jnp.float32)
                nx = jnp.where(fx > tx, fx - 1.0, fx)
                ny = jnp.where(fy > ty, fy - 1.0, fy)
                nz = jnp.where(fz > tz, fz - 1.0, fz)
                ux = _bf16r(dsx - nx)
                uy = _bf16r(dsy - ny)
                uz = _bf16r(dsz - nz)
                px = ux * bx[0] + uy * bx[3] + uz * bx[6]
                py = ux * bx[1] + uy * bx[4] + uz * bx[7]
                pz = ux * bx[2] + uy * bx[5] + uz * bx[8]
                r2 = px * px + py * py + pz * pz

                ibits = plsc.bitcast(r2, jnp.int32)
                ibits = 1597463007 - lax.shift_right_logical(ibits, 1)
                y = plsc.bitcast(ibits, jnp.float32)
                r2h = 0.5 * r2
                y = y * (1.5 - r2h * y * y)
                y = y * (1.5 - r2h * y * y)
                y = y * (1.5 - r2h * y * y)

                term = sq * dq * (y - icut)
                a = a + jnp.where(r2 <= cut2, term, 0.0)
            return a

        return lax.fori_loop(0, CHUNKS_PER_BLK, chunk, acc)

    acc = lax.fori_loop(0, blocks, block, jnp.zeros((L,), jnp.float32))
    acc_v[...] = acc
    pltpu.sync_copy(acc_v, out.at[wid])


def kernel(coords, pairs, box, charges, prefac, cutoff):
    n_atoms = coords.shape[0]
    n_pairs = pairs.shape[0]
    assert (2 * n_pairs) % (NW * IDX_PER_CHUNK) == 0
    chunks_per_tile = 2 * n_pairs // (NW * IDX_PER_CHUNK)
    assert chunks_per_tile % CHUNKS_PER_BLK == 0

    tab = jnp.concatenate(
        [coords.astype(jnp.float32), charges.astype(jnp.float32)[:, None]], axis=1
    ).reshape(-1)
    idx = pairs.astype(jnp.int32).reshape(-1)
    def bf16r(x):
        b = lax.bitcast_convert_type(x, jnp.int32)
        lsb = lax.bitwise_and(lax.shift_right_logical(b, 16), 1)
        r = b + (lsb + 32767)
        r = lax.bitwise_and(r, jnp.int32(-65536))
        return lax.bitcast_convert_type(r, jnp.float32)

    boxinv = bf16r(jnp.linalg.inv(box.astype(jnp.float32)))
    boxq = bf16r(box.astype(jnp.float32))
    cutoff = cutoff.astype(jnp.float32)
    par = jnp.concatenate(
        [
            boxinv.reshape(-1),
            boxq.reshape(-1),
            jnp.stack([cutoff * cutoff, 1.0 / cutoff]),
            jnp.zeros((4,), jnp.float32),
        ]
    )

    mesh = plsc.VectorSubcoreMesh(
        core_axis_name="c", subcore_axis_name="s", num_cores=NC, num_subcores=NS
    )
    run = pl.kernel(
        functools.partial(_body, chunks_per_tile, 4 * n_atoms),
        out_type=jax.ShapeDtypeStruct((NW, L), jnp.float32),
        mesh=mesh,
        compiler_params=pltpu.CompilerParams(needs_layout_passes=False),
        scratch_types=[
            pltpu.VMEM_SHARED((4 * n_atoms,), jnp.float32),
            pltpu.VMEM((24,), jnp.float32),
            pltpu.VMEM((BLK_IDX,), jnp.int32),
            pltpu.VMEM((BLK_EXP,), jnp.int32),
            pltpu.VMEM((BLK_EXP,), jnp.float32),
            pltpu.VMEM((L,), jnp.float32),
            pltpu.SemaphoreType.DMA,
        ],
    )
    partials = run(tab, idx, par)
    return jnp.sum(partials) * prefac.astype(jnp.float32)


# speedup vs baseline: 19.6846x; 1.6339x over previous
import functools

import jax
import jax.numpy as jnp
from jax import lax
from jax.experimental import pallas as pl
from jax.experimental.pallas import tpu as pltpu
from jax.experimental.pallas import tpu_sc as plsc

NC = 2
NS = 16
NW = NC * NS
L = 16

IDX_PER_CHUNK = 128
PAIRS_PER_CHUNK = IDX_PER_CHUNK // 2
CHUNKS_PER_BLK = 25
SEG = PAIRS_PER_CHUNK
BLK_IDX = CHUNKS_PER_BLK * IDX_PER_CHUNK
BLK_EXP = 4 * BLK_IDX


def _bf16r(x):
    b = plsc.bitcast(x, jnp.int32)
    lsb = lax.bitwise_and(lax.shift_right_logical(b, 16), 1)
    r = b + (lsb + 32767)
    r = lax.bitwise_and(r, jnp.int32(-65536))
    return plsc.bitcast(r, jnp.float32)


def _body(chunks_per_tile, tab, idx, par, out,
          tab_sh, par_v, idx_v, exp_v, rows_v, acc_v, gsem, isem):
    cid = lax.axis_index("c")
    sid = lax.axis_index("s")
    wid = sid * NC + cid

    @pl.when(sid == 0)
    def _():
        pltpu.sync_copy(tab, tab_sh)

    pltpu.sync_copy(par, par_v)
    plsc.subcore_barrier()

    plo = par_v[pl.ds(0, L)]
    phi = par_v[pl.ds(8, L)]
    pe = lambda i: plo[i] if i < L else phi[i - 8]
    bi = [pe(i) for i in range(9)]
    bx = [pe(9 + i) for i in range(9)]
    cut2 = pe(18)
    icut = pe(19)

    iota = lax.iota(jnp.int32, L)
    iota2 = iota * 2

    blocks = chunks_per_tile // CHUNKS_PER_BLK
    tile_idx0 = wid * chunks_per_tile * IDX_PER_CHUNK

    def idx_copy(b, buf, sync):
        src = idx.at[pl.ds(tile_idx0 + b * BLK_IDX, BLK_IDX)]
        dst = idx_v.at[pl.ds(buf * BLK_IDX, BLK_IDX)]
        if sync:
            pltpu.sync_copy(src, dst)
        else:
            pltpu.async_copy(src, dst, isem.at[buf])

    def idx_wait(buf):
        pltpu.make_async_copy(
            idx.at[pl.ds(0, BLK_IDX)],
            idx_v.at[pl.ds(buf * BLK_IDX, BLK_IDX)],
            isem.at[buf],
        ).wait()

    def build_exp(buf):
        ib = buf * BLK_IDX
        eb0 = buf * BLK_EXP

        def build(k, carry):
            for h in range(PAIRS_PER_CHUNK // L):
                off = ib + k * IDX_PER_CHUNK + 2 * L * h
                asrc = plsc.load_gather(idx_v, [iota2 + off])
                adst = plsc.load_gather(idx_v, [iota2 + (off + 1)])
                s4 = asrc * 4
                d4 = adst * 4
                eb = eb0 + k * (8 * SEG) + L * h
                exp_v[pl.ds(eb + 0 * SEG, L)] = s4
                exp_v[pl.ds(eb + 1 * SEG, L)] = d4
                exp_v[pl.ds(eb + 2 * SEG, L)] = s4 + 1
                exp_v[pl.ds(eb + 3 * SEG, L)] = d4 + 1
                exp_v[pl.ds(eb + 4 * SEG, L)] = s4 + 2
                exp_v[pl.ds(eb + 5 * SEG, L)] = d4 + 2
                exp_v[pl.ds(eb + 6 * SEG, L)] = s4 + 3
                exp_v[pl.ds(eb + 7 * SEG, L)] = d4 + 3
            return carry

        lax.fori_loop(0, CHUNKS_PER_BLK, build, 0)

    def gather_fire(buf):
        pltpu.async_copy(
            tab_sh.at[exp_v.at[pl.ds(buf * BLK_EXP, BLK_EXP)]],
            rows_v.at[pl.ds(buf * BLK_EXP, BLK_EXP)],
            gsem.at[buf],
        )

    def gather_wait(buf):
        pltpu.make_async_copy(
            tab_sh.at[exp_v.at[pl.ds(buf * BLK_EXP, BLK_EXP)]],
            rows_v.at[pl.ds(buf * BLK_EXP, BLK_EXP)],
            gsem.at[buf],
        ).wait()

    def compute(buf, acc):
        rb0 = buf * BLK_EXP

        def chunk(k, a):
            cb = rb0 + k * (8 * SEG)
            for g in range(PAIRS_PER_CHUNK // L):
                gb = L * g
                sx = rows_v[pl.ds(cb + 0 * SEG + gb, L)]
                dx = rows_v[pl.ds(cb + 1 * SEG + gb, L)]
                sy = rows_v[pl.ds(cb + 2 * SEG + gb, L)]
                dy = rows_v[pl.ds(cb + 3 * SEG + gb, L)]
                sz = rows_v[pl.ds(cb + 4 * SEG + gb, L)]
                dz = rows_v[pl.ds(cb + 5 * SEG + gb, L)]
                sq = rows_v[pl.ds(cb + 6 * SEG + gb, L)]
                dq = rows_v[pl.ds(cb + 7 * SEG + gb, L)]

                qx = _bf16r(sx - dx)
                qy = _bf16r(sy - dy)
                qz = _bf16r(sz - dz)
                dsx = qx * bi[0] + qy * bi[3] + qz * bi[6]
                dsy = qx * bi[1] + qy * bi[4] + qz * bi[7]
                dsz = qx * bi[2] + qy * bi[5] + qz * bi[8]
                tx = dsx + 0.5
                ty = dsy + 0.5
                tz = dsz + 0.5
                fx = tx.astype(jnp.int32).astype(jnp.float32)
                fy = ty.astype(jnp.int32).astype(jnp.float32)
                fz = tz.astype(jnp.int32).astype(jnp.float32)
                nx = jnp.where(fx > tx, fx - 1.0, fx)
                ny = jnp.where(fy > ty, fy - 1.0, fy)
                nz = jnp.where(fz > tz, fz - 1.0, fz)
                ux = _bf16r(dsx - nx)
                uy = _bf16r(dsy - ny)
                uz = _bf16r(dsz - nz)
                px = ux * bx[0] + uy * bx[3] + uz * bx[6]
                py = ux * bx[1] + uy * bx[4] + uz * bx[7]
                pz = ux * bx[2] + uy * bx[5] + uz * bx[8]
                r2 = px * px + py * py + pz * pz

                ibits = plsc.bitcast(r2, jnp.int32)
                ibits = 1597463007 - lax.shift_right_logical(ibits, 1)
                y = plsc.bitcast(ibits, jnp.float32)
                r2h = 0.5 * r2
                y = y * (1.5 - r2h * y * y)
                y = y * (1.5 - r2h * y * y)
                y = y * (1.5 - r2h * y * y)

                term = sq * dq * (y - icut)
                a = a + jnp.where(r2 <= cut2, term, 0.0)
            return a

        return lax.fori_loop(0, CHUNKS_PER_BLK, chunk, acc)

    idx_copy(0, 0, sync=True)
    build_exp(0)
    gather_fire(0)
    idx_copy(1, 1, sync=False)

    def block(b, acc):
        d = lax.rem(b, 2)
        nd = 1 - d

        @pl.when(b < blocks - 1)
        def _():
            idx_wait(nd)
            build_exp(nd)
            gather_fire(nd)

        @pl.when(b < blocks - 2)
        def _():
            idx_copy(b + 2, d, sync=False)

        gather_wait(d)
        return compute(d, acc)

    acc = lax.fori_loop(0, blocks, block, jnp.zeros((L,), jnp.float32))
    acc_v[...] = acc
    pltpu.sync_copy(acc_v, out.at[wid])


def kernel(coords, pairs, box, charges, prefac, cutoff):
    n_atoms = coords.shape[0]
    n_pairs = pairs.shape[0]
    assert (2 * n_pairs) % (NW * IDX_PER_CHUNK) == 0
    chunks_per_tile = 2 * n_pairs // (NW * IDX_PER_CHUNK)
    assert chunks_per_tile % CHUNKS_PER_BLK == 0

    tab = jnp.concatenate(
        [coords.astype(jnp.float32), charges.astype(jnp.float32)[:, None]], axis=1
    ).reshape(-1)
    idx = pairs.astype(jnp.int32).reshape(-1)
    def bf16r(x):
        b = lax.bitcast_convert_type(x, jnp.int32)
        lsb = lax.bitwise_and(lax.shift_right_logical(b, 16), 1)
        r = b + (lsb + 32767)
        r = lax.bitwise_and(r, jnp.int32(-65536))
        return lax.bitcast_convert_type(r, jnp.float32)

    boxinv = bf16r(jnp.linalg.inv(box.astype(jnp.float32)))
    boxq = bf16r(box.astype(jnp.float32))
    cutoff = cutoff.astype(jnp.float32)
    par = jnp.concatenate(
        [
            boxinv.reshape(-1),
            boxq.reshape(-1),
            jnp.stack([cutoff * cutoff, 1.0 / cutoff]),
            jnp.zeros((4,), jnp.float32),
        ]
    )

    mesh = plsc.VectorSubcoreMesh(
        core_axis_name="c", subcore_axis_name="s", num_cores=NC, num_subcores=NS
    )
    run = pl.kernel(
        functools.partial(_body, chunks_per_tile),
        out_type=jax.ShapeDtypeStruct((NW, L), jnp.float32),
        mesh=mesh,
        compiler_params=pltpu.CompilerParams(needs_layout_passes=False),
        scratch_types=[
            pltpu.VMEM_SHARED((4 * n_atoms,), jnp.float32),
            pltpu.VMEM((24,), jnp.float32),
            pltpu.VMEM((2 * BLK_IDX,), jnp.int32),
            pltpu.VMEM((2 * BLK_EXP,), jnp.int32),
            pltpu.VMEM((2 * BLK_EXP,), jnp.float32),
            pltpu.VMEM((L,), jnp.float32),
            pltpu.SemaphoreType.DMA((2,)),
            pltpu.SemaphoreType.DMA((2,)),
        ],
    )
    partials = run(tab, idx, par)
    return jnp.sum(partials) * prefac.astype(jnp.float32)
